# Initial kernel scaffold; baseline (speedup 1.0000x reference)
import jax
import jax.numpy as jnp
from jax.experimental import pallas as pl


def kernel(feat, edge_index, etypes):
    raise NotImplementedError("write your pallas kernel here")


# speedup vs baseline: 3.9282x; 3.9282x over previous
import functools

import jax
import jax.numpy as jnp
from jax import lax
from jax.experimental import pallas as pl
from jax.experimental.pallas import tpu as pltpu
from jax.experimental.pallas import tpu_sc as plsc

N_NODES = 10000
N_EDGES = 320000
D_FEAT = 128

CHUNK = 128
N_CHUNKS = N_EDGES // CHUNK
NC = 2
NS = 16
NW = NC * NS
LANES = 16


def _mu_body(x_ref, o_ref):
    x = x_ref[...]
    o_ref[...] = jnp.exp(-0.5 * x * x)


def _node_mu(feat):
    return pl.pallas_call(
        _mu_body,
        out_shape=jax.ShapeDtypeStruct((N_NODES, D_FEAT), jnp.float32),
        grid=(10,),
        in_specs=[pl.BlockSpec((N_NODES // 10, D_FEAT), lambda i: (i, 0))],
        out_specs=pl.BlockSpec((N_NODES // 10, D_FEAT), lambda i: (i, 0)),
    )(feat)


_mesh = plsc.VectorSubcoreMesh(core_axis_name="c", subcore_axis_name="s")


@functools.partial(
    pl.kernel,
    mesh=_mesh,
    out_type=jax.ShapeDtypeStruct((N_EDGES, D_FEAT), jnp.float32),
    scratch_types=[
        pltpu.VMEM((CHUNK,), jnp.int32),
        pltpu.VMEM((CHUNK,), jnp.int32),
        pltpu.VMEM((CHUNK, D_FEAT), jnp.float32),
        pltpu.VMEM((CHUNK, D_FEAT), jnp.float32),
        pltpu.SemaphoreType.DMA,
        pltpu.SemaphoreType.DMA,
    ],
)
def _edge_min_kernel(mu_hbm, src_hbm, dst_hbm, out_hbm,
                     sidx, didx, bufa, bufb, sema, semb):
    wid = lax.axis_index("s") * NC + lax.axis_index("c")

    def chunk_body(t, carry):
        c = wid + NW * t

        @pl.when(c < N_CHUNKS)
        def _():
            pltpu.sync_copy(src_hbm.at[c], sidx)
            pltpu.sync_copy(dst_hbm.at[c], didx)
            cpa = pltpu.async_copy(mu_hbm.at[sidx], bufa, sema)
            cpb = pltpu.async_copy(mu_hbm.at[didx], bufb, semb)
            cpa.wait()
            cpb.wait()

            def row_body(e, carry2):
                for j in range(D_FEAT // LANES):
                    sl = pl.ds(j * LANES, LANES)
                    bufa[e, sl] = jnp.minimum(bufa[e, sl], bufb[e, sl])
                return carry2

            lax.fori_loop(0, CHUNK, row_body, 0)
            pltpu.sync_copy(bufa, out_hbm.at[pl.ds(c * CHUNK, CHUNK)])

        return carry

    lax.fori_loop(0, (N_CHUNKS + NW - 1) // NW, chunk_body, 0)


def kernel(feat, edge_index, etypes):
    mu = _node_mu(feat)
    src = edge_index[0].astype(jnp.int32).reshape(N_CHUNKS, CHUNK)
    dst = edge_index[1].astype(jnp.int32).reshape(N_CHUNKS, CHUNK)
    return _edge_min_kernel(mu, src, dst)
